# Initial kernel scaffold; baseline (speedup 1.0000x reference)
import functools

import jax
import jax.numpy as jnp
from jax import lax
from jax.experimental import pallas as pl
from jax.experimental.pallas import tpu as pltpu
from jax.experimental.pallas import tpu_sc as plsc

N = 10000
E = 320000
F = 128
NEG_SLOPE = 0.2

NC = 2
NS = 16
NW = NC * NS
EPT = E // NW
C = 80
NCHUNK = EPT // C
GROUPS = C // 16
RPT = N // NS
DZ = 640


def _fc_body(x_ref, w_ref, al_ref, ar_ref, h_ref, el_ref, er_ref):
    h = lax.dot_general(
        x_ref[...], w_ref[...], (((1,), (1,)), ((), ())),
        preferred_element_type=jnp.float32,
        precision=lax.Precision.HIGHEST,
    )
    h_ref[...] = h
    el_ref[...] = jnp.sum(h * al_ref[...], axis=1)
    er_ref[...] = jnp.sum(h * ar_ref[...], axis=1)


def _combine_body(rstp_ref, denp_ref, bias_ref, out_ref):
    s = rstp_ref[0] + rstp_ref[1]
    dn = denp_ref[0] + denp_ref[1]
    dn = jnp.where(dn != 0.0, dn, 1.0)
    out_ref[...] = s / dn[:, None] + bias_ref[...][None, :]


def _edge_body(h_hbm, el_hbm, er_hbm, ei_hbm, rstp_hbm, denp_hbm,
               el_v, er_v, srcg_v, dstg_v, ee_v, rows_v, zden_v):
    cid = lax.axis_index("c")
    sid = lax.axis_index("s")
    wid = cid * NS + sid

    def alloc(rst_sh, den_sh):
        pltpu.sync_copy(el_hbm, el_v)
        pltpu.sync_copy(er_hbm, er_v)
        pltpu.sync_copy(ei_hbm.at[0, wid], srcg_v)
        pltpu.sync_copy(ei_hbm.at[1, wid], dstg_v)

        zeros16 = jnp.zeros((16,), jnp.float32)

        @pl.loop(0, C)
        def _(r):
            for c in range(8):
                rows_v[r, pl.ds(c * 16, 16)] = zeros16

        base = sid * RPT
        for t in range(7):
            pltpu.sync_copy(rows_v, rst_sh.at[pl.ds(base + t * C, C)])
        pltpu.sync_copy(rows_v.at[pl.ds(0, RPT - 7 * C)],
                        rst_sh.at[pl.ds(base + 7 * C, RPT - 7 * C)])

        for t in range(DZ // 16):
            zden_v[pl.ds(t * 16, 16)] = zeros16

        @pl.when(sid < NS - 1)
        def _():
            pltpu.sync_copy(zden_v, den_sh.at[pl.ds(sid * DZ, DZ)])

        @pl.when(sid == NS - 1)
        def _():
            pltpu.sync_copy(zden_v.at[pl.ds(0, N - (NS - 1) * DZ)],
                            den_sh.at[pl.ds((NS - 1) * DZ, N - (NS - 1) * DZ)])

        plsc.subcore_barrier()

        @pl.loop(0, NCHUNK)
        def _(i):
            pltpu.sync_copy(h_hbm.at[srcg_v.at[i]], rows_v)

            for g in range(GROUPS):
                srcv = srcg_v[i, pl.ds(g * 16, 16)]
                dstv = dstg_v[i, pl.ds(g * 16, 16)]
                el_g = plsc.load_gather(el_v, [srcv])
                er_g = plsc.load_gather(er_v, [dstv])
                e = el_g + er_g
                e = jnp.where(e > 0.0, e, e * NEG_SLOPE)
                ee_v[pl.ds(g * 16, 16)] = jnp.exp(e)

            for j in range(C):
                a = plsc.load_gather(ee_v, [jnp.full((16,), j, jnp.int32)])
                for c in range(8):
                    sl = pl.ds(c * 16, 16)
                    rows_v[j, sl] = rows_v[j, sl] * a

            pltpu.sync_copy(rows_v, rst_sh.at[dstg_v.at[i]], add=True)
            pltpu.sync_copy(ee_v, den_sh.at[dstg_v.at[i]], add=True)

        plsc.subcore_barrier()

        obase = sid * RPT
        pltpu.sync_copy(rst_sh.at[pl.ds(obase, RPT)],
                        rstp_hbm.at[cid, pl.ds(obase, RPT)])

        @pl.when(sid < NS - 1)
        def _():
            pltpu.sync_copy(den_sh.at[pl.ds(sid * DZ, DZ)],
                            denp_hbm.at[cid, pl.ds(sid * DZ, DZ)])

        @pl.when(sid == NS - 1)
        def _():
            pltpu.sync_copy(den_sh.at[pl.ds((NS - 1) * DZ, N - (NS - 1) * DZ)],
                            denp_hbm.at[cid, pl.ds((NS - 1) * DZ, N - (NS - 1) * DZ)])

    pl.run_scoped(
        alloc,
        pltpu.VMEM_SHARED((N, F), jnp.float32),
        pltpu.VMEM_SHARED((N,), jnp.float32),
    )


def kernel(x, edge_index, W, attn_l, attn_r, bias):
    al = attn_l.reshape(1, F)
    ar = attn_r.reshape(1, F)

    h, el, er = pl.pallas_call(
        _fc_body,
        out_shape=(
            jax.ShapeDtypeStruct((N, F), jnp.float32),
            jax.ShapeDtypeStruct((N,), jnp.float32),
            jax.ShapeDtypeStruct((N,), jnp.float32),
        ),
    )(x, W, al, ar)

    ei = edge_index.reshape(2, NW, NCHUNK, C)

    mesh = plsc.VectorSubcoreMesh(core_axis_name="c", subcore_axis_name="s")
    edge_kernel = pl.kernel(
        _edge_body,
        out_type=(
            jax.ShapeDtypeStruct((NC, N, F), jnp.float32),
            jax.ShapeDtypeStruct((NC, N), jnp.float32),
        ),
        mesh=mesh,
        scratch_types=[
            pltpu.VMEM((N,), jnp.float32),
            pltpu.VMEM((N,), jnp.float32),
            pltpu.VMEM((NCHUNK, C), jnp.int32),
            pltpu.VMEM((NCHUNK, C), jnp.int32),
            pltpu.VMEM((C,), jnp.float32),
            pltpu.VMEM((C, F), jnp.float32),
            pltpu.VMEM((DZ,), jnp.float32),
        ],
    )
    rstp, denp = edge_kernel(h, el, er, ei)

    out = pl.pallas_call(
        _combine_body,
        out_shape=jax.ShapeDtypeStruct((N, F), jnp.float32),
    )(rstp, denp, bias)
    return out


# speedup vs baseline: 14.3244x; 14.3244x over previous
import dataclasses

import jax
import jax.numpy as jnp
from jax import lax
from jax.experimental import pallas as pl
from jax.experimental.pallas import tpu as pltpu
from jax.experimental.pallas import tpu_sc as plsc

N = 10000
E = 320000
F = 128
NEG_SLOPE = 0.2

NC = 2
NS = 16
NW = NC * NS
EPT = E // NW
EPTP = 10240
C = 128
NCHUNK = EPTP // C
GROUPS = C // 16
DZ = 640
NP = 10240


def _fc_body(x_ref, w_ref, al_ref, ar_ref, h_ref, el_ref, er_ref):
    h = lax.dot_general(
        x_ref[...], w_ref[...], (((1,), (1,)), ((), ())),
        preferred_element_type=jnp.float32,
        precision=lax.Precision.HIGHEST,
    )
    h_ref[...] = h
    el_ref[pl.ds(0, N)] = jnp.sum(h * al_ref[...], axis=1)
    el_ref[pl.ds(N, NP - N)] = jnp.zeros((NP - N,), jnp.float32)
    er_ref[pl.ds(0, N)] = jnp.sum(h * ar_ref[...], axis=1)
    er_ref[pl.ds(N, NP - N)] = jnp.zeros((NP - N,), jnp.float32)


def _combine_body(rstp_ref, denp_ref, bias_ref, out_ref):
    s = rstp_ref[0] + rstp_ref[1]
    dn = (denp_ref[0] + denp_ref[1])[:N]
    dn = jnp.where(dn != 0.0, dn, 1.0)
    out_ref[...] = s / dn[:, None] + bias_ref[...][None, :]


def _edge_body(h_hbm, el_hbm, er_hbm, ei_hbm, rstp_hbm, denp_hbm,
               srcg_v, dstg_v, elg_v, erg_v, ee_v, rows_v, zden_v,
               rst_sh, den_sh):
    cid = lax.axis_index("c")
    sid = lax.axis_index("s")
    wid = cid * NS + sid

    pltpu.sync_copy(ei_hbm.at[0, wid], srcg_v)
    pltpu.sync_copy(ei_hbm.at[1, wid], dstg_v)

    zeros16 = jnp.zeros((16,), jnp.float32)

    @pl.loop(0, C)
    def _(r):
        for c in range(8):
            rows_v[r, pl.ds(c * 16, 16)] = zeros16

    base = sid * DZ
    nstrips = jnp.where(sid < NS - 1, DZ // C, (N - (NS - 1) * DZ) // C)

    @pl.loop(0, nstrips)
    def _(t):
        pltpu.sync_copy(rows_v, rst_sh.at[pl.ds(base + t * C, C)])

    @pl.when(sid == NS - 1)
    def _():
        pltpu.sync_copy(rows_v.at[pl.ds(0, 16)],
                        rst_sh.at[pl.ds((NS - 1) * DZ + 384, 16)])

    for t in range(DZ // 16):
        zden_v[pl.ds(t * 16, 16)] = zeros16
    pltpu.sync_copy(zden_v, den_sh.at[pl.ds(sid * DZ, DZ)])

    plsc.subcore_barrier()

    @pl.loop(0, NCHUNK)
    def _(i):
        src_idx = srcg_v.at[i]
        dst_idx = dstg_v.at[i]
        pltpu.sync_copy(el_hbm.at[src_idx], elg_v)
        pltpu.sync_copy(er_hbm.at[dst_idx], erg_v)
        pltpu.sync_copy(h_hbm.at[src_idx], rows_v)

        for g in range(GROUPS):
            sl = pl.ds(g * 16, 16)
            e = elg_v[sl] + erg_v[sl]
            e = jnp.where(e > 0.0, e, e * NEG_SLOPE)
            ee = jnp.exp(e)
            pos = i * C + g * 16 + lax.iota(jnp.int32, 16)
            ee_v[pl.ds(16 + g * 16, 16)] = jnp.where(pos < EPT, ee, 0.0)

        for j in range(C):
            a = plsc.load_gather(ee_v, [jnp.full((16,), 16 + j, jnp.int32)])
            for c in range(8):
                sl = pl.ds(c * 16, 16)
                rows_v[j, sl] = rows_v[j, sl] * a

        pltpu.sync_copy(rows_v, rst_sh.at[dst_idx], add=True)
        pltpu.sync_copy(ee_v.at[pl.ds(16, C)], den_sh.at[dst_idx], add=True)

    plsc.subcore_barrier()

    pltpu.sync_copy(den_sh.at[pl.ds(sid * DZ, DZ)],
                    denp_hbm.at[cid, pl.ds(sid * DZ, DZ)])

    @pl.when(sid < NS - 1)
    def _():
        pltpu.sync_copy(rst_sh.at[pl.ds(sid * DZ, DZ)],
                        rstp_hbm.at[cid, pl.ds(sid * DZ, DZ)])

    @pl.when(sid == NS - 1)
    def _():
        last = N - (NS - 1) * DZ
        pltpu.sync_copy(rst_sh.at[pl.ds((NS - 1) * DZ, last)],
                        rstp_hbm.at[cid, pl.ds((NS - 1) * DZ, last)])


def kernel(x, edge_index, W, attn_l, attn_r, bias):
    al = attn_l.reshape(1, F)
    ar = attn_r.reshape(1, F)

    h, el, er = pl.pallas_call(
        _fc_body,
        out_shape=(
            jax.ShapeDtypeStruct((N, F), jnp.float32),
            jax.ShapeDtypeStruct((NP,), jnp.float32),
            jax.ShapeDtypeStruct((NP,), jnp.float32),
        ),
    )(x, W, al, ar)

    ei = jnp.pad(edge_index.reshape(2, NW, EPT),
                 ((0, 0), (0, 0), (0, EPTP - EPT))).reshape(2, NW, NCHUNK, C)

    mesh = plsc.VectorSubcoreMesh(core_axis_name="c", subcore_axis_name="s",
                                  num_cores=NC, num_subcores=NS)
    cp = pltpu.CompilerParams()
    if "needs_layout_passes" in pltpu.CompilerParams.__dataclass_fields__:
        cp = dataclasses.replace(cp, needs_layout_passes=False)
    edge_kernel = pl.kernel(
        _edge_body,
        out_type=(
            jax.ShapeDtypeStruct((NC, N, F), jnp.float32),
            jax.ShapeDtypeStruct((NC, NP), jnp.float32),
        ),
        mesh=mesh,
        scratch_types=[
            pltpu.VMEM((NCHUNK, C), jnp.int32),
            pltpu.VMEM((NCHUNK, C), jnp.int32),
            pltpu.VMEM((C,), jnp.float32),
            pltpu.VMEM((C,), jnp.float32),
            pltpu.VMEM((C + 16,), jnp.float32),
            pltpu.VMEM((C, F), jnp.float32),
            pltpu.VMEM((DZ,), jnp.float32),
            pltpu.VMEM_SHARED((N, F), jnp.float32),
            pltpu.VMEM_SHARED((NP,), jnp.float32),
        ],
        compiler_params=cp,
    )
    rstp, denp = edge_kernel(h, el, er, ei)

    out = pl.pallas_call(
        _combine_body,
        out_shape=jax.ShapeDtypeStruct((N, F), jnp.float32),
    )(rstp, denp, bias)
    return out


# speedup vs baseline: 20.2389x; 1.4129x over previous
import dataclasses

import jax
import jax.numpy as jnp
from jax import lax
from jax.experimental import pallas as pl
from jax.experimental.pallas import tpu as pltpu
from jax.experimental.pallas import tpu_sc as plsc

N = 10000
E = 320000
F = 128
NEG_SLOPE = 0.2

NC = 2
NS = 16
NW = NC * NS
EPT = E // NW
EPTP = 10240
C = 64
NPAIR = EPTP // (2 * C)
NCHUNK = EPTP // C
GROUPS = C // 16
DZ = 640
NP = 10240


def _fc_body(x_ref, w_ref, al_ref, ar_ref, h_ref, el_ref, er_ref):
    h = lax.dot_general(
        x_ref[...], w_ref[...], (((1,), (1,)), ((), ())),
        preferred_element_type=jnp.float32,
        precision=lax.Precision.HIGHEST,
    )
    h_ref[...] = h
    el_ref[pl.ds(0, N)] = jnp.sum(h * al_ref[...], axis=1)
    el_ref[pl.ds(N, NP - N)] = jnp.zeros((NP - N,), jnp.float32)
    er_ref[pl.ds(0, N)] = jnp.sum(h * ar_ref[...], axis=1)
    er_ref[pl.ds(N, NP - N)] = jnp.zeros((NP - N,), jnp.float32)


def _combine_body(rstp_ref, denp_ref, bias_ref, out_ref):
    s = rstp_ref[0] + rstp_ref[1]
    dn = (denp_ref[0] + denp_ref[1])[:N]
    dn = jnp.where(dn != 0.0, dn, 1.0)
    out_ref[...] = s / dn[:, None] + bias_ref[...][None, :]


def _edge_body(h_hbm, el_hbm, er_hbm, ei_hbm, rstp_hbm, denp_hbm,
               packed_v, rows_g, rows_s, elg_v, erg_v, ee_v,
               src_s, dst_g, dst_s, zden_v,
               gsem0, gsem1, ssem0, ssem1,
               rst_sh, den_sh):
    cid = lax.axis_index("c")
    sid = lax.axis_index("s")
    wid = cid * NS + sid
    gsem = (gsem0, gsem1)
    ssem = (ssem0, ssem1)

    pltpu.sync_copy(ei_hbm.at[wid], packed_v)

    zeros16 = jnp.zeros((16,), jnp.float32)

    @pl.loop(0, C)
    def _(r):
        for c in range(8):
            rows_g[0, r, pl.ds(c * 16, 16)] = zeros16

    base = sid * DZ
    nstrips = jnp.where(sid < NS - 1, DZ // C, 400 // C)

    @pl.loop(0, nstrips)
    def _(t):
        pltpu.sync_copy(rows_g.at[0], rst_sh.at[pl.ds(base + t * C, C)])

    @pl.when(sid == NS - 1)
    def _():
        pltpu.sync_copy(rows_g.at[0, pl.ds(0, 16)],
                        rst_sh.at[pl.ds((NS - 1) * DZ + 384, 16)])

    for t in range(DZ // 16):
        zden_v[pl.ds(t * 16, 16)] = zeros16
    pltpu.sync_copy(zden_v, den_sh.at[pl.ds(sid * DZ, DZ)])

    plsc.subcore_barrier()

    def unpack(row, half, b):
        for g in range(GROUPS):
            p = packed_v[row, pl.ds(half * C + g * 16, 16)]
            src_s[b, pl.ds(g * 16, 16)] = lax.shift_right_logical(p, 14)
            dst_g[b, pl.ds(g * 16, 16)] = lax.bitwise_and(p, 16383)

    def issue_gathers(b):
        pltpu.async_copy(el_hbm.at[src_s.at[b]], elg_v.at[b], gsem[b])
        pltpu.async_copy(er_hbm.at[dst_g.at[b]], erg_v.at[b], gsem[b])
        pltpu.async_copy(h_hbm.at[src_s.at[b]], rows_g.at[b], gsem[b])

    def drain_gathers(b):
        pltpu.make_async_copy(el_hbm.at[src_s.at[b]], elg_v.at[b], gsem[b]).wait()
        pltpu.make_async_copy(er_hbm.at[dst_g.at[b]], erg_v.at[b], gsem[b]).wait()
        pltpu.make_async_copy(h_hbm.at[src_s.at[b]], rows_g.at[b], gsem[b]).wait()

    def issue_scatters(b):
        pltpu.async_copy(rows_s.at[b], rst_sh.at[dst_s.at[b]], ssem[b], add=True)
        pltpu.async_copy(ee_v.at[b, pl.ds(16, C)], den_sh.at[dst_s.at[b]],
                         ssem[b], add=True)

    def drain_scatters(b):
        pltpu.make_async_copy(rows_s.at[b], rst_sh.at[dst_s.at[b]], ssem[b]).wait()
        pltpu.make_async_copy(ee_v.at[b, pl.ds(16, C)], den_sh.at[dst_s.at[b]],
                              ssem[b]).wait()

    unpack(0, 0, 0)
    issue_gathers(0)
    unpack(0, 1, 1)
    issue_gathers(1)

    @pl.loop(0, NPAIR)
    def _(r):
        for half in (0, 1):
            b = half
            c_dyn = r * 2 + half
            drain_gathers(b)

            @pl.when(c_dyn >= 2)
            def _():
                drain_scatters(b)

            for g in range(GROUPS):
                sl = pl.ds(g * 16, 16)
                e = elg_v[b, sl] + erg_v[b, sl]
                e = jnp.where(e > 0.0, e, e * NEG_SLOPE)
                ee = jnp.exp(e)
                pos = c_dyn * C + g * 16 + lax.iota(jnp.int32, 16)
                ee_v[b, pl.ds(16 + g * 16, 16)] = jnp.where(pos < EPT, ee, 0.0)
                dst_s[b, sl] = dst_g[b, sl]

            for j in range(C):
                a = plsc.load_gather(ee_v.at[b],
                                     [jnp.full((16,), 16 + j, jnp.int32)])
                for cc in range(8):
                    sl = pl.ds(cc * 16, 16)
                    rows_s[b, j, sl] = rows_g[b, j, sl] * a

            issue_scatters(b)

            @pl.when(r < NPAIR - 1)
            def _():
                unpack(r + 1, half, b)
                issue_gathers(b)

    drain_scatters(0)
    drain_scatters(1)

    plsc.subcore_barrier()

    pltpu.sync_copy(den_sh.at[pl.ds(sid * DZ, DZ)],
                    denp_hbm.at[cid, pl.ds(sid * DZ, DZ)])

    @pl.when(sid < NS - 1)
    def _():
        pltpu.sync_copy(rst_sh.at[pl.ds(sid * DZ, DZ)],
                        rstp_hbm.at[cid, pl.ds(sid * DZ, DZ)])

    @pl.when(sid == NS - 1)
    def _():
        last = N - (NS - 1) * DZ
        pltpu.sync_copy(rst_sh.at[pl.ds((NS - 1) * DZ, last)],
                        rstp_hbm.at[cid, pl.ds((NS - 1) * DZ, last)])


def kernel(x, edge_index, W, attn_l, attn_r, bias):
    al = attn_l.reshape(1, F)
    ar = attn_r.reshape(1, F)

    h, el, er = pl.pallas_call(
        _fc_body,
        out_shape=(
            jax.ShapeDtypeStruct((N, F), jnp.float32),
            jax.ShapeDtypeStruct((NP,), jnp.float32),
            jax.ShapeDtypeStruct((NP,), jnp.float32),
        ),
    )(x, W, al, ar)

    eir = jnp.pad(edge_index.reshape(2, NW, EPT),
                  ((0, 0), (0, 0), (0, EPTP - EPT)))
    packed = (eir[0] << 14) | eir[1]
    packed = packed.reshape(NW, EPTP // 128, 128)

    mesh = plsc.VectorSubcoreMesh(core_axis_name="c", subcore_axis_name="s",
                                  num_cores=NC, num_subcores=NS)
    cp = pltpu.CompilerParams()
    if "needs_layout_passes" in pltpu.CompilerParams.__dataclass_fields__:
        cp = dataclasses.replace(cp, needs_layout_passes=False)
    edge_kernel = pl.kernel(
        _edge_body,
        out_type=(
            jax.ShapeDtypeStruct((NC, N, F), jnp.float32),
            jax.ShapeDtypeStruct((NC, NP), jnp.float32),
        ),
        mesh=mesh,
        scratch_types=[
            pltpu.VMEM((EPTP // 128, 128), jnp.int32),
            pltpu.VMEM((2, C, F), jnp.float32),
            pltpu.VMEM((2, C, F), jnp.float32),
            pltpu.VMEM((2, C), jnp.float32),
            pltpu.VMEM((2, C), jnp.float32),
            pltpu.VMEM((2, C + 16), jnp.float32),
            pltpu.VMEM((2, C), jnp.int32),
            pltpu.VMEM((2, C), jnp.int32),
            pltpu.VMEM((2, C), jnp.int32),
            pltpu.VMEM((DZ,), jnp.float32),
            pltpu.SemaphoreType.DMA,
            pltpu.SemaphoreType.DMA,
            pltpu.SemaphoreType.DMA,
            pltpu.SemaphoreType.DMA,
            pltpu.VMEM_SHARED((N, F), jnp.float32),
            pltpu.VMEM_SHARED((NP,), jnp.float32),
        ],
        compiler_params=cp,
    )
    rstp, denp = edge_kernel(h, el, er, packed)

    out = pl.pallas_call(
        _combine_body,
        out_shape=jax.ShapeDtypeStruct((N, F), jnp.float32),
    )(rstp, denp, bias)
    return out


# speedup vs baseline: 22.9609x; 1.1345x over previous
import dataclasses

import jax
import jax.numpy as jnp
from jax import lax
from jax.experimental import pallas as pl
from jax.experimental.pallas import tpu as pltpu
from jax.experimental.pallas import tpu_sc as plsc

N = 10000
E = 320000
F = 128
NEG_SLOPE = 0.2

NC = 2
NS = 16
NW = NC * NS
EPT = E // NW
EPTP = 10240
C = 64
NPAIR = EPTP // (2 * C)
NCHUNK = EPTP // C
GROUPS = C // 16
DZ = 640
NP = 10240


def _fc_body(x_ref, w_ref, al_ref, ar_ref, h_ref, el_ref, er_ref):
    h = lax.dot_general(
        x_ref[...], w_ref[...], (((1,), (1,)), ((), ())),
        preferred_element_type=jnp.float32,
        precision=lax.Precision.HIGHEST,
    )
    h_ref[...] = h
    el_ref[pl.ds(0, N)] = jnp.sum(h * al_ref[...], axis=1)
    el_ref[pl.ds(N, NP - N)] = jnp.zeros((NP - N,), jnp.float32)
    er_ref[pl.ds(0, N)] = jnp.sum(h * ar_ref[...], axis=1)
    er_ref[pl.ds(N, NP - N)] = jnp.zeros((NP - N,), jnp.float32)


def _combine_body(rstp_ref, denp_ref, bias_ref, out_ref):
    s = rstp_ref[0] + rstp_ref[1]
    dn = (denp_ref[0] + denp_ref[1])[:N]
    dn = jnp.where(dn != 0.0, dn, 1.0)
    out_ref[...] = s / dn[:, None] + bias_ref[...][None, :]


def _edge_body(h_hbm, el_hbm, er_hbm, ei_hbm, rstp_hbm, denp_hbm,
               packed_v, rows_g, rows_s, elg_v, erg_v, ee_v,
               src_s, dst_g, dst_s, zden_v,
               gsem0, gsem1, ssem0, ssem1,
               rst_sh, den_sh):
    cid = lax.axis_index("c")
    sid = lax.axis_index("s")
    wid = cid * NS + sid
    gsem = (gsem0, gsem1)
    ssem = (ssem0, ssem1)

    pltpu.sync_copy(ei_hbm.at[wid], packed_v)

    zeros16 = jnp.zeros((16,), jnp.float32)

    @pl.loop(0, C)
    def _(r):
        for c in range(8):
            rows_g[0, r, pl.ds(c * 16, 16)] = zeros16

    base = sid * DZ
    nstrips = jnp.where(sid < NS - 1, DZ // C, 400 // C)

    @pl.loop(0, nstrips)
    def _(t):
        pltpu.sync_copy(rows_g.at[0], rst_sh.at[pl.ds(base + t * C, C)])

    @pl.when(sid == NS - 1)
    def _():
        pltpu.sync_copy(rows_g.at[0, pl.ds(0, 16)],
                        rst_sh.at[pl.ds((NS - 1) * DZ + 384, 16)])

    for t in range(DZ // 16):
        zden_v[pl.ds(t * 16, 16)] = zeros16
    pltpu.sync_copy(zden_v, den_sh.at[pl.ds(sid * DZ, DZ)])

    plsc.subcore_barrier()

    def unpack(row, half, b):
        for g in range(GROUPS):
            p = packed_v[row, pl.ds(half * C + g * 16, 16)]
            src_s[b, pl.ds(g * 16, 16)] = lax.shift_right_logical(p, 14)
            dst_g[b, pl.ds(g * 16, 16)] = lax.bitwise_and(p, 16383)

    def issue_gathers(b):
        pltpu.async_copy(el_hbm.at[src_s.at[b]], elg_v.at[b], gsem[b])
        pltpu.async_copy(er_hbm.at[dst_g.at[b]], erg_v.at[b], gsem[b])
        pltpu.async_copy(h_hbm.at[src_s.at[b]], rows_g.at[b], gsem[b])

    def drain_gathers(b):
        pltpu.make_async_copy(el_hbm.at[src_s.at[b]], elg_v.at[b], gsem[b]).wait()
        pltpu.make_async_copy(er_hbm.at[dst_g.at[b]], erg_v.at[b], gsem[b]).wait()
        pltpu.make_async_copy(h_hbm.at[src_s.at[b]], rows_g.at[b], gsem[b]).wait()

    def issue_scatters(b):
        pltpu.async_copy(rows_s.at[b], rst_sh.at[dst_s.at[b]], ssem[b], add=True)
        pltpu.async_copy(ee_v.at[b, pl.ds(16, C)], den_sh.at[dst_s.at[b]],
                         ssem[b], add=True)

    def drain_scatters(b):
        pltpu.make_async_copy(rows_s.at[b], rst_sh.at[dst_s.at[b]], ssem[b]).wait()
        pltpu.make_async_copy(ee_v.at[b, pl.ds(16, C)], den_sh.at[dst_s.at[b]],
                              ssem[b]).wait()

    unpack(0, 0, 0)
    issue_gathers(0)
    unpack(0, 1, 1)
    issue_gathers(1)

    @pl.loop(0, NPAIR)
    def _(r):
        for half in (0, 1):
            b = half
            c_dyn = r * 2 + half
            drain_gathers(b)

            @pl.when(c_dyn >= 2)
            def _():
                drain_scatters(b)

            for g in range(GROUPS):
                sl = pl.ds(g * 16, 16)
                e = elg_v[b, sl] + erg_v[b, sl]
                e = jnp.where(e > 0.0, e, e * NEG_SLOPE)
                ee = jnp.exp(e)
                pos = c_dyn * C + g * 16 + lax.iota(jnp.int32, 16)
                ee_v[b, pl.ds(16 + g * 16, 16)] = jnp.where(pos < EPT, ee, 0.0)
                dst_s[b, sl] = dst_g[b, sl]

            gdn = lax.GatherDimensionNumbers(
                offset_dims=(), collapsed_slice_dims=(0,), start_index_map=(0,))
            for g in range(GROUPS):
                eeg = ee_v[b, pl.ds(16 + g * 16, 16)]
                for jj in range(16):
                    j = g * 16 + jj
                    a = lax.gather(
                        eeg, jnp.full((16, 1), jj, jnp.int32), gdn, (1,),
                        mode=lax.GatherScatterMode.PROMISE_IN_BOUNDS)
                    for cc in range(8):
                        sl = pl.ds(cc * 16, 16)
                        rows_s[b, j, sl] = rows_g[b, j, sl] * a

            issue_scatters(b)

            @pl.when(r < NPAIR - 1)
            def _():
                unpack(r + 1, half, b)
                issue_gathers(b)

    drain_scatters(0)
    drain_scatters(1)

    plsc.subcore_barrier()

    pltpu.sync_copy(den_sh.at[pl.ds(sid * DZ, DZ)],
                    denp_hbm.at[cid, pl.ds(sid * DZ, DZ)])

    @pl.when(sid < NS - 1)
    def _():
        pltpu.sync_copy(rst_sh.at[pl.ds(sid * DZ, DZ)],
                        rstp_hbm.at[cid, pl.ds(sid * DZ, DZ)])

    @pl.when(sid == NS - 1)
    def _():
        last = N - (NS - 1) * DZ
        pltpu.sync_copy(rst_sh.at[pl.ds((NS - 1) * DZ, last)],
                        rstp_hbm.at[cid, pl.ds((NS - 1) * DZ, last)])


def kernel(x, edge_index, W, attn_l, attn_r, bias):
    al = attn_l.reshape(1, F)
    ar = attn_r.reshape(1, F)

    h, el, er = pl.pallas_call(
        _fc_body,
        out_shape=(
            jax.ShapeDtypeStruct((N, F), jnp.float32),
            jax.ShapeDtypeStruct((NP,), jnp.float32),
            jax.ShapeDtypeStruct((NP,), jnp.float32),
        ),
    )(x, W, al, ar)

    eir = jnp.pad(edge_index.reshape(2, NW, EPT),
                  ((0, 0), (0, 0), (0, EPTP - EPT)))
    packed = (eir[0] << 14) | eir[1]
    packed = packed.reshape(NW, EPTP // 128, 128)

    mesh = plsc.VectorSubcoreMesh(core_axis_name="c", subcore_axis_name="s",
                                  num_cores=NC, num_subcores=NS)
    cp = pltpu.CompilerParams()
    if "needs_layout_passes" in pltpu.CompilerParams.__dataclass_fields__:
        cp = dataclasses.replace(cp, needs_layout_passes=False)
    edge_kernel = pl.kernel(
        _edge_body,
        out_type=(
            jax.ShapeDtypeStruct((NC, N, F), jnp.float32),
            jax.ShapeDtypeStruct((NC, NP), jnp.float32),
        ),
        mesh=mesh,
        scratch_types=[
            pltpu.VMEM((EPTP // 128, 128), jnp.int32),
            pltpu.VMEM((2, C, F), jnp.float32),
            pltpu.VMEM((2, C, F), jnp.float32),
            pltpu.VMEM((2, C), jnp.float32),
            pltpu.VMEM((2, C), jnp.float32),
            pltpu.VMEM((2, C + 16), jnp.float32),
            pltpu.VMEM((2, C), jnp.int32),
            pltpu.VMEM((2, C), jnp.int32),
            pltpu.VMEM((2, C), jnp.int32),
            pltpu.VMEM((DZ,), jnp.float32),
            pltpu.SemaphoreType.DMA,
            pltpu.SemaphoreType.DMA,
            pltpu.SemaphoreType.DMA,
            pltpu.SemaphoreType.DMA,
            pltpu.VMEM_SHARED((N, F), jnp.float32),
            pltpu.VMEM_SHARED((NP,), jnp.float32),
        ],
        compiler_params=cp,
    )
    rstp, denp = edge_kernel(h, el, er, packed)

    out = pl.pallas_call(
        _combine_body,
        out_shape=jax.ShapeDtypeStruct((N, F), jnp.float32),
    )(rstp, denp, bias)
    return out


# speedup vs baseline: 22.9648x; 1.0002x over previous
import dataclasses

import jax
import jax.numpy as jnp
from jax import lax
from jax.experimental import pallas as pl
from jax.experimental.pallas import tpu as pltpu
from jax.experimental.pallas import tpu_sc as plsc

N = 10000
E = 320000
F = 128
NEG_SLOPE = 0.2

NC = 2
NS = 16
NW = NC * NS
EPT = E // NW
EPTP = 10240
C = 64
NPAIR = EPTP // (2 * C)
NCHUNK = EPTP // C
GROUPS = C // 16
DZ = 640
NP = 10240


def _fc_body(x_ref, w_ref, al_ref, ar_ref, h_ref, el_ref, er_ref):
    h = lax.dot_general(
        x_ref[...], w_ref[...], (((1,), (1,)), ((), ())),
        preferred_element_type=jnp.float32,
        precision=lax.Precision.HIGHEST,
    )
    h_ref[...] = h
    el_ref[pl.ds(0, N)] = jnp.sum(h * al_ref[...], axis=1)
    el_ref[pl.ds(N, NP - N)] = jnp.zeros((NP - N,), jnp.float32)
    er_ref[pl.ds(0, N)] = jnp.sum(h * ar_ref[...], axis=1)
    er_ref[pl.ds(N, NP - N)] = jnp.zeros((NP - N,), jnp.float32)


def _combine_body(rstp_ref, denp_ref, bias_ref, out_ref):
    s = rstp_ref[0] + rstp_ref[1]
    dn = (denp_ref[0] + denp_ref[1])[:N]
    dn = jnp.where(dn != 0.0, dn, 1.0)
    out_ref[...] = s / dn[:, None] + bias_ref[...][None, :]


def _edge_body(h_hbm, el_hbm, er_hbm, ei_hbm, rstp_hbm, denp_hbm,
               packed_v, rows_g, rows_s, elg_v, erg_v, ee_v,
               src_s, dst_g, dst_s, zden_v,
               gsem0, gsem1, ssem0, ssem1,
               rst_sh, den_sh):
    cid = lax.axis_index("c")
    sid = lax.axis_index("s")
    wid = cid * NS + sid
    gsem = (gsem0, gsem1)
    ssem = (ssem0, ssem1)

    pltpu.sync_copy(ei_hbm.at[wid], packed_v)

    zeros16 = jnp.zeros((16,), jnp.float32)

    @pl.loop(0, C)
    def _(r):
        for c in range(8):
            rows_g[0, r, pl.ds(c * 16, 16)] = zeros16

    base = sid * DZ
    nstrips = jnp.where(sid < NS - 1, DZ // C, 400 // C)

    @pl.loop(0, nstrips)
    def _(t):
        pltpu.sync_copy(rows_g.at[0], rst_sh.at[pl.ds(base + t * C, C)])

    @pl.when(sid == NS - 1)
    def _():
        pltpu.sync_copy(rows_g.at[0, pl.ds(0, 16)],
                        rst_sh.at[pl.ds((NS - 1) * DZ + 384, 16)])

    for t in range(DZ // 16):
        zden_v[pl.ds(t * 16, 16)] = zeros16
    pltpu.sync_copy(zden_v, den_sh.at[pl.ds(sid * DZ, DZ)])

    plsc.subcore_barrier()

    def unpack(row, half, b):
        for g in range(GROUPS):
            p = packed_v[row, pl.ds(half * C + g * 16, 16)]
            src_s[b, pl.ds(g * 16, 16)] = lax.shift_right_logical(p, 14)
            dst_g[b, pl.ds(g * 16, 16)] = lax.bitwise_and(p, 16383)

    def issue_gathers(b):
        pltpu.async_copy(el_hbm.at[src_s.at[b]], elg_v.at[b], gsem[b])
        pltpu.async_copy(er_hbm.at[dst_g.at[b]], erg_v.at[b], gsem[b])
        pltpu.async_copy(h_hbm.at[src_s.at[b, pl.ds(0, C // 2)]],
                         rows_g.at[b, pl.ds(0, C // 2)], gsem[b])
        pltpu.async_copy(h_hbm.at[src_s.at[b, pl.ds(C // 2, C // 2)]],
                         rows_g.at[b, pl.ds(C // 2, C // 2)], gsem[b])

    def drain_gathers(b):
        pltpu.make_async_copy(el_hbm.at[src_s.at[b]], elg_v.at[b], gsem[b]).wait()
        pltpu.make_async_copy(er_hbm.at[dst_g.at[b]], erg_v.at[b], gsem[b]).wait()
        pltpu.make_async_copy(h_hbm.at[src_s.at[b, pl.ds(0, C // 2)]],
                              rows_g.at[b, pl.ds(0, C // 2)], gsem[b]).wait()
        pltpu.make_async_copy(h_hbm.at[src_s.at[b, pl.ds(C // 2, C // 2)]],
                              rows_g.at[b, pl.ds(C // 2, C // 2)], gsem[b]).wait()

    def issue_scatters(b):
        pltpu.async_copy(rows_s.at[b], rst_sh.at[dst_s.at[b]], ssem[b], add=True)
        pltpu.async_copy(ee_v.at[b, pl.ds(16, C)], den_sh.at[dst_s.at[b]],
                         ssem[b], add=True)

    def drain_scatters(b):
        pltpu.make_async_copy(rows_s.at[b], rst_sh.at[dst_s.at[b]], ssem[b]).wait()
        pltpu.make_async_copy(ee_v.at[b, pl.ds(16, C)], den_sh.at[dst_s.at[b]],
                              ssem[b]).wait()

    unpack(0, 0, 0)
    issue_gathers(0)
    unpack(0, 1, 1)
    issue_gathers(1)

    @pl.loop(0, NPAIR)
    def _(r):
        for half in (0, 1):
            b = half
            c_dyn = r * 2 + half
            drain_gathers(b)

            @pl.when(c_dyn >= 2)
            def _():
                drain_scatters(b)

            for g in range(GROUPS):
                sl = pl.ds(g * 16, 16)
                e = elg_v[b, sl] + erg_v[b, sl]
                e = jnp.where(e > 0.0, e, e * NEG_SLOPE)
                ee = jnp.exp(e)
                pos = c_dyn * C + g * 16 + lax.iota(jnp.int32, 16)
                ee_v[b, pl.ds(16 + g * 16, 16)] = jnp.where(pos < EPT, ee, 0.0)
                dst_s[b, sl] = dst_g[b, sl]

            gdn = lax.GatherDimensionNumbers(
                offset_dims=(), collapsed_slice_dims=(0,), start_index_map=(0,))
            for g in range(GROUPS):
                eeg = ee_v[b, pl.ds(16 + g * 16, 16)]
                for jj in range(16):
                    j = g * 16 + jj
                    a = lax.gather(
                        eeg, jnp.full((16, 1), jj, jnp.int32), gdn, (1,),
                        mode=lax.GatherScatterMode.PROMISE_IN_BOUNDS)
                    for cc in range(8):
                        sl = pl.ds(cc * 16, 16)
                        rows_s[b, j, sl] = rows_g[b, j, sl] * a

            issue_scatters(b)

            @pl.when(r < NPAIR - 1)
            def _():
                unpack(r + 1, half, b)
                issue_gathers(b)

    drain_scatters(0)
    drain_scatters(1)

    plsc.subcore_barrier()

    pltpu.sync_copy(den_sh.at[pl.ds(sid * DZ, DZ)],
                    denp_hbm.at[cid, pl.ds(sid * DZ, DZ)])

    @pl.when(sid < NS - 1)
    def _():
        pltpu.sync_copy(rst_sh.at[pl.ds(sid * DZ, DZ)],
                        rstp_hbm.at[cid, pl.ds(sid * DZ, DZ)])

    @pl.when(sid == NS - 1)
    def _():
        last = N - (NS - 1) * DZ
        pltpu.sync_copy(rst_sh.at[pl.ds((NS - 1) * DZ, last)],
                        rstp_hbm.at[cid, pl.ds((NS - 1) * DZ, last)])


def kernel(x, edge_index, W, attn_l, attn_r, bias):
    al = attn_l.reshape(1, F)
    ar = attn_r.reshape(1, F)

    h, el, er = pl.pallas_call(
        _fc_body,
        out_shape=(
            jax.ShapeDtypeStruct((N, F), jnp.float32),
            jax.ShapeDtypeStruct((NP,), jnp.float32),
            jax.ShapeDtypeStruct((NP,), jnp.float32),
        ),
    )(x, W, al, ar)

    eir = jnp.pad(edge_index.reshape(2, NW, EPT),
                  ((0, 0), (0, 0), (0, EPTP - EPT)))
    packed = (eir[0] << 14) | eir[1]
    packed = packed.reshape(NW, EPTP // 128, 128)

    mesh = plsc.VectorSubcoreMesh(core_axis_name="c", subcore_axis_name="s",
                                  num_cores=NC, num_subcores=NS)
    cp = pltpu.CompilerParams()
    if "needs_layout_passes" in pltpu.CompilerParams.__dataclass_fields__:
        cp = dataclasses.replace(cp, needs_layout_passes=False)
    edge_kernel = pl.kernel(
        _edge_body,
        out_type=(
            jax.ShapeDtypeStruct((NC, N, F), jnp.float32),
            jax.ShapeDtypeStruct((NC, NP), jnp.float32),
        ),
        mesh=mesh,
        scratch_types=[
            pltpu.VMEM((EPTP // 128, 128), jnp.int32),
            pltpu.VMEM((2, C, F), jnp.float32),
            pltpu.VMEM((2, C, F), jnp.float32),
            pltpu.VMEM((2, C), jnp.float32),
            pltpu.VMEM((2, C), jnp.float32),
            pltpu.VMEM((2, C + 16), jnp.float32),
            pltpu.VMEM((2, C), jnp.int32),
            pltpu.VMEM((2, C), jnp.int32),
            pltpu.VMEM((2, C), jnp.int32),
            pltpu.VMEM((DZ,), jnp.float32),
            pltpu.SemaphoreType.DMA,
            pltpu.SemaphoreType.DMA,
            pltpu.SemaphoreType.DMA,
            pltpu.SemaphoreType.DMA,
            pltpu.VMEM_SHARED((N, F), jnp.float32),
            pltpu.VMEM_SHARED((NP,), jnp.float32),
        ],
        compiler_params=cp,
    )
    rstp, denp = edge_kernel(h, el, er, packed)

    out = pl.pallas_call(
        _combine_body,
        out_shape=jax.ShapeDtypeStruct((N, F), jnp.float32),
    )(rstp, denp, bias)
    return out
